# baseline (speedup 1.0000x reference)
import functools

import jax
import jax.numpy as jnp
from jax import lax
from jax.experimental import pallas as pl
from jax.experimental.pallas import tpu as pltpu
from jax.experimental.pallas import tpu_sc as plsc

_F = 26
_V = 104000
_D = 16
_B = 4096
_C = 4000
_NC = 2
_NS = 16
_NW = _NC * _NS
_NBLK = _F * _F
_HD = 8
_W = 4096
_TASK = _HD * _B
_NP = 2
_PIECE = _TASK // _NP
_TAILC = 100000


def _params(wid, t):
    blk = wid + _NW * (t // 2)
    f, tt = blk // _F, blk % _F
    c0 = jnp.where(f == _F - 1, 0, ((f * _C) // 128) * 128)
    off = f * _C - jnp.where(f == _F - 1, _TAILC, c0)
    return f, tt, c0, off


def _lookup_kernel(
    inT_hbm, tpl_hbm, tail_hbm, out_hbm, idx_v, win_v, obuf_v, ism, wsm, osm
):
    wid = lax.axis_index("s") * _NC + lax.axis_index("c")
    nvalid = jnp.where(wid < _NBLK - 21 * _NW, 44, 42)

    def win_copies(b, t):
        f, tt, c0, _ = _params(wid, t)
        rows = pl.ds(tt * _D + (t % 2) * _HD, _HD)
        main = pltpu.make_async_copy(
            tpl_hbm.at[rows, pl.ds(c0, _W)], win_v.at[b], wsm.at[b]
        )
        tail = pltpu.make_async_copy(tail_hbm.at[rows, :], win_v.at[b], wsm.at[b])
        return f, main, tail

    def issue_in(b, t):
        f, main, tail = win_copies(b, t)
        pltpu.async_copy(
            inT_hbm.at[pl.ds(f * (_B // 128), _B // 128), :], idx_v.at[b], ism.at[b]
        )

        @pl.when(f == _F - 1)
        def _():
            tail.start()

        @pl.when(f != _F - 1)
        def _():
            main.start()

    def wait_in(b, t):
        f, main, tail = win_copies(b, t)
        pltpu.make_async_copy(
            inT_hbm.at[pl.ds(f * (_B // 128), _B // 128), :], idx_v.at[b], ism.at[b]
        ).wait()

        @pl.when(f == _F - 1)
        def _():
            tail.wait()

        @pl.when(f != _F - 1)
        def _():
            main.wait()

    for b in range(2):
        issue_in(b, b)

    def body(k, carry):
        for b in range(2):
            t = 2 * k + b
            blk = wid + _NW * k
            f, tt, c0, off = _params(wid, t)
            rbase = blk * (2 * _TASK // 128) + b * (_TASK // 128)

            @pl.when(t < nvalid)
            def _():
                wait_in(b, t)

                offv = jnp.full((16,), off, jnp.int32)
                dvecs = [jnp.full((16,), d8, jnp.int32) for d8 in range(_HD)]

                for p in range(_NP):
                    dst = out_hbm.at[
                        pl.ds(rbase + p * (_PIECE // 128), _PIECE // 128), :
                    ]

                    def drain():
                        pltpu.make_async_copy(obuf_v.at[p], dst, osm.at[p]).wait()

                    if b == 1:
                        drain()
                    else:

                        @pl.when(k >= 1)
                        def _dr():
                            drain()

                    @plsc.parallel_loop(0, _PIECE // 1024, 1)
                    def v_step(i):
                        for j in range(8):
                            iv = idx_v.at[b][
                                p * (_PIECE // 1024) + i, pl.ds(j * 16, 16)
                            ]
                            c = iv + offv
                            for d8 in range(_HD):
                                obuf_v.at[p][
                                    i * 8 + d8, pl.ds(j * 16, 16)
                                ] = plsc.load_gather(win_v.at[b], [dvecs[d8], c])

                    pltpu.async_copy(obuf_v.at[p], dst, osm.at[p])

                @pl.when(t + 2 < nvalid)
                def _prefetch():
                    issue_in(b, t + 2)

        return carry

    lax.fori_loop(0, 22, body, 0)

    for s in range(_NP):
        pltpu.make_async_copy(
            obuf_v.at[s], out_hbm.at[pl.ds(0, _PIECE // 128), :], osm.at[s]
        ).wait()


@jax.jit
def _run(inT, tpl, tail):
    mesh = plsc.VectorSubcoreMesh(
        core_axis_name="c", subcore_axis_name="s", num_cores=_NC, num_subcores=_NS
    )
    f = functools.partial(
        pl.kernel,
        mesh=mesh,
        out_type=jax.ShapeDtypeStruct((_NBLK * 2 * _TASK // 128, 128), jnp.float32),
        scratch_types=[
            pltpu.VMEM((2, _B // 128, 128), jnp.int32),
            pltpu.VMEM((2, _HD, _W), jnp.float32),
            pltpu.VMEM((_NP, _PIECE // 128, 128), jnp.float32),
            pltpu.SemaphoreType.DMA((2,)),
            pltpu.SemaphoreType.DMA((2,)),
            pltpu.SemaphoreType.DMA((_NP,)),
        ],
        compiler_params=pltpu.CompilerParams(
            use_tc_tiling_on_sc=True, needs_layout_passes=False
        ),
    )(_lookup_kernel)
    return f(inT, tpl, tail)


def kernel(inputs, tables):
    inT = jnp.transpose(inputs, (1, 0)).reshape(_F * _B // 128, 128)
    tpl = jnp.transpose(tables, (0, 2, 1)).reshape(_F * _D, _V)
    tail = jnp.transpose(tables[:, _TAILC:, :], (0, 2, 1)).reshape(_F * _D, _C)
    tail = jnp.pad(tail, ((0, 0), (0, 96)))
    flat = _run(inT, tpl, tail)
    out6 = flat.reshape(_F, _F, 2, _B // 128, _HD, 128)
    out = jnp.transpose(out6, (3, 5, 0, 1, 2, 4))
    return out.reshape(_B, _F, _F, _D)


# speedup vs baseline: 1.0476x; 1.0476x over previous
import functools

import jax
import jax.numpy as jnp
from jax import lax
from jax.experimental import pallas as pl
from jax.experimental.pallas import tpu as pltpu
from jax.experimental.pallas import tpu_sc as plsc

_F = 26
_V = 104000
_D = 16
_B = 4096
_C = 4000
_NC = 2
_NS = 16
_NW = _NC * _NS
_NBLK = _F * _F
_HD = 8
_W = 4096
_TASK = _HD * _B
_NP = 2
_PIECE = _TASK // _NP
_TAILC = 100000


def _params(wid, t):
    blk = wid + _NW * (t // 2)
    f, tt = blk // _F, blk % _F
    c0 = jnp.where(f == _F - 1, 0, ((f * _C) // 128) * 128)
    off = f * _C - jnp.where(f == _F - 1, _TAILC, c0)
    return f, tt, c0, off


def _lookup_kernel(
    inT_hbm, tpl_hbm, tail_hbm, out_hbm, idx_v, win_v, obuf_v, inT_sh, ism, wsm, osm
):
    wid = lax.axis_index("s") * _NC + lax.axis_index("c")
    nvalid = jnp.where(wid < _NBLK - 21 * _NW, 44, 42)

    @pl.when(lax.axis_index("s") == 0)
    def _stage():
        pltpu.sync_copy(inT_hbm, inT_sh)

    plsc.subcore_barrier()

    def win_copies(b, t):
        f, tt, c0, _ = _params(wid, t)
        rows = pl.ds(tt * _D + (t % 2) * _HD, _HD)
        main = pltpu.make_async_copy(
            tpl_hbm.at[rows, pl.ds(c0, _W)], win_v.at[b], wsm.at[b]
        )
        tail = pltpu.make_async_copy(tail_hbm.at[rows, :], win_v.at[b], wsm.at[b])
        return f, main, tail

    def issue_in(b, t):
        f, main, tail = win_copies(b, t)
        pltpu.async_copy(
            inT_sh.at[pl.ds(f * (_B // 128), _B // 128), :], idx_v.at[b], ism.at[b]
        )

        @pl.when(f == _F - 1)
        def _():
            tail.start()

        @pl.when(f != _F - 1)
        def _():
            main.start()

    def wait_in(b, t):
        f, main, tail = win_copies(b, t)
        pltpu.make_async_copy(
            inT_sh.at[pl.ds(f * (_B // 128), _B // 128), :], idx_v.at[b], ism.at[b]
        ).wait()

        @pl.when(f == _F - 1)
        def _():
            tail.wait()

        @pl.when(f != _F - 1)
        def _():
            main.wait()

    for b in range(2):
        issue_in(b, b)

    def body(k, carry):
        for b in range(2):
            t = 2 * k + b
            blk = wid + _NW * k
            f, tt, c0, off = _params(wid, t)
            rbase = blk * (2 * _TASK // 128) + b * (_TASK // 128)

            @pl.when(t < nvalid)
            def _():
                wait_in(b, t)

                offv = jnp.full((16,), off, jnp.int32)
                dvecs = [jnp.full((16,), d8, jnp.int32) for d8 in range(_HD)]

                for p in range(_NP):
                    dst = out_hbm.at[
                        pl.ds(rbase + p * (_PIECE // 128), _PIECE // 128), :
                    ]

                    def drain():
                        pltpu.make_async_copy(obuf_v.at[p], dst, osm.at[p]).wait()

                    if b == 1:
                        drain()
                    else:

                        @pl.when(k >= 1)
                        def _dr():
                            drain()

                    @plsc.parallel_loop(0, _PIECE // 1024, 1)
                    def v_step(i):
                        for j in range(8):
                            iv = idx_v.at[b][
                                p * (_PIECE // 1024) + i, pl.ds(j * 16, 16)
                            ]
                            c = iv + offv
                            for d8 in range(_HD):
                                obuf_v.at[p][
                                    i * 8 + d8, pl.ds(j * 16, 16)
                                ] = plsc.load_gather(win_v.at[b], [dvecs[d8], c])

                    pltpu.async_copy(obuf_v.at[p], dst, osm.at[p])

                @pl.when(t + 2 < nvalid)
                def _prefetch():
                    issue_in(b, t + 2)

        return carry

    lax.fori_loop(0, 22, body, 0)

    for s in range(_NP):
        pltpu.make_async_copy(
            obuf_v.at[s], out_hbm.at[pl.ds(0, _PIECE // 128), :], osm.at[s]
        ).wait()


@jax.jit
def _run(inT, tpl, tail):
    mesh = plsc.VectorSubcoreMesh(
        core_axis_name="c", subcore_axis_name="s", num_cores=_NC, num_subcores=_NS
    )
    f = functools.partial(
        pl.kernel,
        mesh=mesh,
        out_type=jax.ShapeDtypeStruct((_NBLK * 2 * _TASK // 128, 128), jnp.float32),
        scratch_types=[
            pltpu.VMEM((2, _B // 128, 128), jnp.int32),
            pltpu.VMEM((2, _HD, _W), jnp.float32),
            pltpu.VMEM((_NP, _PIECE // 128, 128), jnp.float32),
            pltpu.VMEM_SHARED((_F * _B // 128, 128), jnp.int32),
            pltpu.SemaphoreType.DMA((2,)),
            pltpu.SemaphoreType.DMA((2,)),
            pltpu.SemaphoreType.DMA((_NP,)),
        ],
        compiler_params=pltpu.CompilerParams(
            use_tc_tiling_on_sc=True, needs_layout_passes=False
        ),
    )(_lookup_kernel)
    return f(inT, tpl, tail)


def kernel(inputs, tables):
    inT = jnp.transpose(inputs, (1, 0)).reshape(_F * _B // 128, 128)
    tpl = jnp.transpose(tables, (0, 2, 1)).reshape(_F * _D, _V)
    tail = jnp.transpose(tables[:, _TAILC:, :], (0, 2, 1)).reshape(_F * _D, _C)
    tail = jnp.pad(tail, ((0, 0), (0, 96)))
    flat = _run(inT, tpl, tail)
    out6 = flat.reshape(_F, _F, 2, _B // 128, _HD, 128)
    out = jnp.transpose(out6, (3, 5, 0, 1, 2, 4))
    return out.reshape(_B, _F, _F, _D)
